# Initial kernel scaffold; baseline (speedup 1.0000x reference)
import jax
import jax.numpy as jnp
from jax.experimental import pallas as pl


def kernel(quant_ids, quant_embedding_weight):
    raise NotImplementedError("write your pallas kernel here")


# speedup vs baseline: 1.5435x; 1.5435x over previous
import functools

import jax
import jax.numpy as jnp
from jax import lax
from jax.experimental import pallas as pl
from jax.experimental.pallas import tpu as pltpu
from jax.experimental.pallas import tpu_sc as plsc

N_ROWS = 204800
D = 144
TD = 24
NC, NS = 2, 16
NW = NC * NS
ROWS_PER_W = N_ROWS // NW
CHUNK = 128
N_CHUNKS = ROWS_PER_W // CHUNK

_LOG1P = (
    1.47206501e-06,
    0.999847697,
    -0.497373216,
    0.315747317,
    -0.190354337,
    0.0826912371,
    -0.0174140775,
)


def _softplus16(x):
    e = jnp.exp(-jnp.abs(x))
    p = jnp.full_like(e, _LOG1P[6])
    for c in (_LOG1P[5], _LOG1P[4], _LOG1P[3], _LOG1P[2], _LOG1P[1], _LOG1P[0]):
        p = p * e + c
    return jnp.maximum(x, 0.0) + p


def _sigmoid16(x):
    e = jnp.exp(-jnp.abs(x))
    num = jnp.where(x >= 0.0, jnp.ones_like(e), e)
    return num / (1.0 + e)


def _body(ids_hbm, tab_hbm,
          x_hbm, qs_hbm, dz_hbm, rh_hbm, th_hbm, rs_hbm, ts_hbm,
          idx_v, raw_v, act_v, gsem):
    wid = lax.axis_index("s") * NC + lax.axis_index("c")
    base0 = wid * ROWS_PER_W

    def chunk_body(g, carry):
        base = base0 + g * CHUNK
        pltpu.sync_copy(ids_hbm.at[pl.ds(base, CHUNK)], idx_v)
        pltpu.async_copy(tab_hbm.at[idx_v], raw_v, gsem).wait()
        pltpu.sync_copy(raw_v, x_hbm.at[pl.ds(base, CHUNK)])

        def row_body(j, c2):
            for v in range(9):
                xv = raw_v[j, pl.ds(v * 16, 16)]
                yv = _softplus16(xv) if v < 3 else _sigmoid16(xv)
                act_v[j, pl.ds(v * 16, 16)] = yv
            return c2

        lax.fori_loop(0, CHUNK, row_body, 0, unroll=2)

        for k, out in enumerate((qs_hbm, dz_hbm, ts_hbm, rs_hbm, th_hbm, rh_hbm)):
            pltpu.sync_copy(act_v.at[:, pl.ds(k * TD, TD)],
                            out.at[pl.ds(base, CHUNK)])
        return carry

    lax.fori_loop(0, N_CHUNKS, chunk_body, 0)


_mesh = plsc.VectorSubcoreMesh(core_axis_name="c", subcore_axis_name="s")
_out_type = (
    jax.ShapeDtypeStruct((N_ROWS, D), jnp.float32),
) + tuple(jax.ShapeDtypeStruct((N_ROWS, TD), jnp.float32) for _ in range(6))
_scratch = [
    pltpu.VMEM((CHUNK,), jnp.int32),
    pltpu.VMEM((CHUNK, D), jnp.float32),
    pltpu.VMEM((CHUNK, D), jnp.float32),
    pltpu.SemaphoreType.DMA,
]

_sc_call = pl.kernel(_body, out_type=_out_type, mesh=_mesh,
                     scratch_types=_scratch,
                     compiler_params=pltpu.CompilerParams(
                         use_tc_tiling_on_sc=False))


def kernel(quant_ids, quant_embedding_weight):
    B, S = quant_ids.shape
    ids = quant_ids.reshape(-1).astype(jnp.int32)
    x, qs, dz, rh, th, rs, ts = _sc_call(ids, quant_embedding_weight)
    r = lambda a: a.reshape(B, S, a.shape[-1])
    return (r(x), r(qs), r(dz), r(rh), r(th), r(rs), r(ts))


# speedup vs baseline: 2.3709x; 1.5361x over previous
import jax
import jax.numpy as jnp
from jax import lax
from jax.experimental import pallas as pl
from jax.experimental.pallas import tpu as pltpu
from jax.experimental.pallas import tpu_sc as plsc

N_ROWS = 204800
D = 144
TD = 24
NC, NS = 2, 16
NW = NC * NS
ROWS_PER_W = N_ROWS // NW
CHUNK = 128
N_CHUNKS = ROWS_PER_W // CHUNK

_LOG1P = (
    1.47206501e-06,
    0.999847697,
    -0.497373216,
    0.315747317,
    -0.190354337,
    0.0826912371,
    -0.0174140775,
)


def _softplus16(x):
    e = jnp.exp(-jnp.abs(x))
    p = jnp.full_like(e, _LOG1P[6])
    for c in (_LOG1P[5], _LOG1P[4], _LOG1P[3], _LOG1P[2], _LOG1P[1], _LOG1P[0]):
        p = p * e + c
    return jnp.maximum(x, 0.0) + p


def _sigmoid16(x):
    e = jnp.exp(-jnp.abs(x))
    num = jnp.where(x >= 0.0, jnp.ones_like(e), e)
    return num / (1.0 + e)


def _body(ids_hbm, tab_hbm,
          x_hbm, qs_hbm, dz_hbm, rh_hbm, th_hbm, rs_hbm, ts_hbm,
          idx_all, raw0, raw1, act0, act1, gs0, gs1, os0, os1):
    wid = lax.axis_index("s") * NC + lax.axis_index("c")
    base0 = wid * ROWS_PER_W
    raws = (raw0, raw1)
    acts = (act0, act1)
    gsems = (gs0, gs1)
    osems = (os0, os1)
    outs = (qs_hbm, dz_hbm, ts_hbm, rs_hbm, th_hbm, rh_hbm)

    pltpu.sync_copy(ids_hbm.at[pl.ds(wid * N_CHUNKS, N_CHUNKS)], idx_all)

    def gather_start(c, b):
        pltpu.async_copy(tab_hbm.at[idx_all.at[c]], raws[b], gsems[b])

    def gather_wait(c, b):
        pltpu.make_async_copy(tab_hbm.at[idx_all.at[c]], raws[b],
                              gsems[b]).wait()

    def compute(b):
        raw_v, act_v = raws[b], acts[b]

        @plsc.parallel_loop(0, CHUNK, unroll=2)
        def row_body(j):
            for v in range(9):
                xv = raw_v[j, pl.ds(v * 16, 16)]
                yv = _softplus16(xv) if v < 3 else _sigmoid16(xv)
                act_v[j, pl.ds(v * 16, 16)] = yv

    def outs_start(c, b):
        base = base0 + c * CHUNK
        pltpu.async_copy(raws[b], x_hbm.at[pl.ds(base, CHUNK)], osems[b])
        for k, out in enumerate(outs):
            pltpu.async_copy(acts[b].at[:, pl.ds(k * TD, TD)],
                             out.at[pl.ds(base, CHUNK)], osems[b])

    def outs_wait(c, b):
        base = base0 + c * CHUNK
        pltpu.make_async_copy(raws[b], x_hbm.at[pl.ds(base, CHUNK)],
                              osems[b]).wait()
        for k, out in enumerate(outs):
            pltpu.make_async_copy(acts[b].at[:, pl.ds(k * TD, TD)],
                                  out.at[pl.ds(base, CHUNK)], osems[b]).wait()

    gather_start(0, 0)

    def two_steps(i, carry):
        for b in (0, 1):
            c = 2 * i + b
            nb = 1 - b

            @pl.when(c >= 1)
            def _():
                outs_wait(c - 1, nb)

            @pl.when(c + 1 < N_CHUNKS)
            def _():
                gather_start(c + 1, nb)

            gather_wait(c, b)
            compute(b)
            outs_start(c, b)
        return carry

    lax.fori_loop(0, N_CHUNKS // 2, two_steps, 0)
    outs_wait(N_CHUNKS - 1, 1)


_mesh = plsc.VectorSubcoreMesh(core_axis_name="c", subcore_axis_name="s")
_out_type = (
    jax.ShapeDtypeStruct((N_ROWS, D), jnp.float32),
) + tuple(jax.ShapeDtypeStruct((N_ROWS, TD), jnp.float32) for _ in range(6))
_scratch = [
    pltpu.VMEM((N_CHUNKS, CHUNK), jnp.int32),
    pltpu.VMEM((CHUNK, D), jnp.float32),
    pltpu.VMEM((CHUNK, D), jnp.float32),
    pltpu.VMEM((CHUNK, D), jnp.float32),
    pltpu.VMEM((CHUNK, D), jnp.float32),
    pltpu.SemaphoreType.DMA,
    pltpu.SemaphoreType.DMA,
    pltpu.SemaphoreType.DMA,
    pltpu.SemaphoreType.DMA,
]

_sc_call = pl.kernel(_body, out_type=_out_type, mesh=_mesh,
                     scratch_types=_scratch,
                     compiler_params=pltpu.CompilerParams(
                         use_tc_tiling_on_sc=False))


def kernel(quant_ids, quant_embedding_weight):
    B, S = quant_ids.shape
    ids = quant_ids.reshape(N_ROWS // CHUNK, CHUNK).astype(jnp.int32)
    x, qs, dz, rh, th, rs, ts = _sc_call(ids, quant_embedding_weight)
    r = lambda a: a.reshape(B, S, a.shape[-1])
    return (r(x), r(qs), r(dz), r(rh), r(th), r(rs), r(ts))


# speedup vs baseline: 5.8880x; 2.4834x over previous
import jax
import jax.numpy as jnp
from jax import lax
from jax.experimental import pallas as pl
from jax.experimental.pallas import tpu as pltpu
from jax.experimental.pallas import tpu_sc as plsc

B = 4096
S = 50
D = 144
TD = 24
N_ROWS = B * S
NC, NS = 2, 16
NW = NC * NS
BW = B // NW
BB = 512


def _sc_body(idst_hbm, tab_hbm, lo_hbm, hi_hbm,
             idx_all, raw0, raw1, gs0, gs1, os0, os1):
    wid = lax.axis_index("s") * NC + lax.axis_index("c")
    b0 = wid * BW
    raws = (raw0, raw1)
    gsems = (gs0, gs1)
    osems = (os0, os1)

    pltpu.sync_copy(idst_hbm.at[:, pl.ds(b0, BW)], idx_all)

    def gstart(s, b):
        pltpu.async_copy(tab_hbm.at[idx_all.at[s]], raws[b], gsems[b])

    def gwait(s, b):
        pltpu.make_async_copy(tab_hbm.at[idx_all.at[s]], raws[b],
                              gsems[b]).wait()

    def ostart(s, b):
        r0 = s * B + b0
        pltpu.async_copy(raws[b].at[:, pl.ds(0, 128)],
                         lo_hbm.at[pl.ds(r0, BW)], osems[b])
        pltpu.async_copy(raws[b].at[:, pl.ds(128, 16)],
                         hi_hbm.at[pl.ds(r0, BW), pl.ds(0, 16)], osems[b])

    def owait(s, b):
        r0 = s * B + b0
        pltpu.make_async_copy(raws[b].at[:, pl.ds(0, 128)],
                              lo_hbm.at[pl.ds(r0, BW)], osems[b]).wait()
        pltpu.make_async_copy(raws[b].at[:, pl.ds(128, 16)],
                              hi_hbm.at[pl.ds(r0, BW), pl.ds(0, 16)],
                              osems[b]).wait()

    gstart(0, 0)

    def two_steps(i, carry):
        for b in (0, 1):
            s = 2 * i + b
            nb = 1 - b

            @pl.when(s >= 1)
            def _():
                owait(s - 1, nb)

            @pl.when(s + 1 < S)
            def _():
                gstart(s + 1, nb)

            gwait(s, b)
            ostart(s, b)
        return carry

    lax.fori_loop(0, S // 2, two_steps, 0)
    owait(S - 1, 1)


_mesh = plsc.VectorSubcoreMesh(core_axis_name="c", subcore_axis_name="s")
_sc_out = (
    jax.ShapeDtypeStruct((N_ROWS, 128), jnp.float32),
    jax.ShapeDtypeStruct((N_ROWS, 128), jnp.float32),
)
_sc_scratch = [
    pltpu.VMEM((S, BW), jnp.int32),
    pltpu.VMEM((BW, D), jnp.float32),
    pltpu.VMEM((BW, D), jnp.float32),
    pltpu.SemaphoreType.DMA,
    pltpu.SemaphoreType.DMA,
    pltpu.SemaphoreType.DMA,
    pltpu.SemaphoreType.DMA,
]
_sc_call = pl.kernel(_sc_body, out_type=_sc_out, mesh=_mesh,
                     scratch_types=_sc_scratch,
                     compiler_params=pltpu.CompilerParams(
                         use_tc_tiling_on_sc=False))


def _tc_body(lo_ref, hi_ref, x_ref, *out_refs):
    lo = lo_ref[...]
    hi = hi_ref[...]
    loT = jnp.transpose(lo, (1, 0))
    hiT = jnp.transpose(hi, (1, 0))
    raw = jnp.concatenate([loT, hiT[:16]], axis=0)
    x_ref[0, :, :] = raw
    sp = jnp.maximum(raw[:48], 0.0) + jnp.log1p(jnp.exp(-jnp.abs(raw[:48])))
    sg = jax.nn.sigmoid(raw[48:])
    act = jnp.concatenate([sp, sg], axis=0)
    for k, o in enumerate(out_refs):
        o[0, :, :] = act[TD * k:TD * (k + 1), :]


def _tc_call(lo, hi):
    grid = (S, B // BB)
    in_spec = pl.BlockSpec((BB, 128), lambda s, j: (s * (B // BB) + j, 0))
    x_spec = pl.BlockSpec((1, D, BB), lambda s, j: (s, 0, j))
    a_spec = pl.BlockSpec((1, TD, BB), lambda s, j: (s, 0, j))
    return pl.pallas_call(
        _tc_body,
        grid=grid,
        in_specs=[in_spec, in_spec],
        out_specs=[x_spec] + [a_spec] * 6,
        out_shape=[jax.ShapeDtypeStruct((S, D, B), jnp.float32)] +
                  [jax.ShapeDtypeStruct((S, TD, B), jnp.float32)] * 6,
    )(lo, hi)


def kernel(quant_ids, quant_embedding_weight):
    ids_t = jnp.transpose(quant_ids).astype(jnp.int32)
    lo, hi = _sc_call(ids_t, quant_embedding_weight)
    x_t, qs, dz, ts, rs, th, rh = _tc_call(lo, hi)
    tr = lambda a: jnp.transpose(a, (2, 0, 1))
    return (tr(x_t), tr(qs), tr(dz), tr(rh), tr(th), tr(rs), tr(ts))


# speedup vs baseline: 10.0285x; 1.7032x over previous
import jax
import jax.numpy as jnp
from jax import lax
from jax.experimental import pallas as pl
from jax.experimental.pallas import tpu as pltpu
from jax.experimental.pallas import tpu_sc as plsc

B = 4096
S = 50
D = 144
TD = 24
V = 100000
N_ROWS = B * S
NC, NS = 2, 16
NW = NC * NS
BW = B // NW
BB = 512
VB = 2048


def _pre_body(tt_ref, lo_ref, hi_ref):
    x = jnp.transpose(tt_ref[...], (1, 0))
    lo_ref[...] = x[:, :128]
    hi_ref[...] = jnp.concatenate(
        [x[:, 128:], jnp.zeros((VB, 128 - (D - 128)), jnp.float32)], axis=1)


def _pre_call(tab_t):
    spec = pl.BlockSpec((VB, 128), lambda i: (i, 0))
    return pl.pallas_call(
        _pre_body,
        grid=(pl.cdiv(V, VB),),
        in_specs=[pl.BlockSpec((D, VB), lambda i: (0, i))],
        out_specs=[spec, spec],
        out_shape=[jax.ShapeDtypeStruct((V, 128), jnp.float32)] * 2,
    )(tab_t)


def _sc_body(idst_hbm, tlo_hbm, thi_hbm, lo_hbm, hi_hbm,
             idx_all, rlo0, rlo1, rhi0, rhi1, gs0, gs1, os0, os1):
    wid = lax.axis_index("s") * NC + lax.axis_index("c")
    b0 = wid * BW
    rlos = (rlo0, rlo1)
    rhis = (rhi0, rhi1)
    gsems = (gs0, gs1)
    osems = (os0, os1)

    pltpu.sync_copy(idst_hbm.at[:, pl.ds(b0, BW)], idx_all)

    def gstart(s, b):
        pltpu.async_copy(tlo_hbm.at[idx_all.at[s]], rlos[b], gsems[b])
        pltpu.async_copy(thi_hbm.at[idx_all.at[s]], rhis[b], gsems[b])

    def gwait(s, b):
        pltpu.make_async_copy(tlo_hbm.at[idx_all.at[s]], rlos[b],
                              gsems[b]).wait()
        pltpu.make_async_copy(thi_hbm.at[idx_all.at[s]], rhis[b],
                              gsems[b]).wait()

    def ostart(s, b):
        r0 = s * B + b0
        pltpu.async_copy(rlos[b], lo_hbm.at[pl.ds(r0, BW)], osems[b])
        pltpu.async_copy(rhis[b], hi_hbm.at[pl.ds(r0, BW)], osems[b])

    def owait(s, b):
        r0 = s * B + b0
        pltpu.make_async_copy(rlos[b], lo_hbm.at[pl.ds(r0, BW)],
                              osems[b]).wait()
        pltpu.make_async_copy(rhis[b], hi_hbm.at[pl.ds(r0, BW)],
                              osems[b]).wait()

    gstart(0, 0)

    def two_steps(i, carry):
        for b in (0, 1):
            s = 2 * i + b
            nb = 1 - b

            @pl.when(s >= 1)
            def _():
                owait(s - 1, nb)

            @pl.when(s + 1 < S)
            def _():
                gstart(s + 1, nb)

            gwait(s, b)
            ostart(s, b)
        return carry

    lax.fori_loop(0, S // 2, two_steps, 0)
    owait(S - 1, 1)


_mesh = plsc.VectorSubcoreMesh(core_axis_name="c", subcore_axis_name="s")
_sc_out = (
    jax.ShapeDtypeStruct((N_ROWS, 128), jnp.float32),
    jax.ShapeDtypeStruct((N_ROWS, 128), jnp.float32),
)
_sc_scratch = [
    pltpu.VMEM((S, BW), jnp.int32),
    pltpu.VMEM((BW, 128), jnp.float32),
    pltpu.VMEM((BW, 128), jnp.float32),
    pltpu.VMEM((BW, 128), jnp.float32),
    pltpu.VMEM((BW, 128), jnp.float32),
    pltpu.SemaphoreType.DMA,
    pltpu.SemaphoreType.DMA,
    pltpu.SemaphoreType.DMA,
    pltpu.SemaphoreType.DMA,
]
_sc_call = pl.kernel(_sc_body, out_type=_sc_out, mesh=_mesh,
                     scratch_types=_sc_scratch,
                     compiler_params=pltpu.CompilerParams(
                         use_tc_tiling_on_sc=False))


def _tc_body(lo_ref, hi_ref, x_ref, *out_refs):
    lo = lo_ref[...]
    hi = hi_ref[...]
    loT = jnp.transpose(lo, (1, 0))
    hiT = jnp.transpose(hi, (1, 0))
    raw = jnp.concatenate([loT, hiT[:16]], axis=0)
    x_ref[0, :, :] = raw
    sp = jnp.maximum(raw[:48], 0.0) + jnp.log1p(jnp.exp(-jnp.abs(raw[:48])))
    sg = jax.nn.sigmoid(raw[48:])
    act = jnp.concatenate([sp, sg], axis=0)
    for k, o in enumerate(out_refs):
        o[0, :, :] = act[TD * k:TD * (k + 1), :]


def _tc_call(lo, hi):
    grid = (S, B // BB)
    in_spec = pl.BlockSpec((BB, 128), lambda s, j: (s * (B // BB) + j, 0))
    x_spec = pl.BlockSpec((1, D, BB), lambda s, j: (s, 0, j))
    a_spec = pl.BlockSpec((1, TD, BB), lambda s, j: (s, 0, j))
    return pl.pallas_call(
        _tc_body,
        grid=grid,
        in_specs=[in_spec, in_spec],
        out_specs=[x_spec] + [a_spec] * 6,
        out_shape=[jax.ShapeDtypeStruct((S, D, B), jnp.float32)] +
                  [jax.ShapeDtypeStruct((S, TD, B), jnp.float32)] * 6,
    )(lo, hi)


def kernel(quant_ids, quant_embedding_weight):
    ids_t = jnp.transpose(quant_ids).astype(jnp.int32)
    tab_lo, tab_hi = _pre_call(jnp.transpose(quant_embedding_weight))
    lo, hi = _sc_call(ids_t, tab_lo, tab_hi)
    x_t, qs, dz, ts, rs, th, rh = _tc_call(lo, hi)
    tr = lambda a: jnp.transpose(a, (2, 0, 1))
    return (tr(x_t), tr(qs), tr(dz), tr(rh), tr(th), tr(rs), tr(ts))


# speedup vs baseline: 12.2276x; 1.2193x over previous
import jax
import jax.numpy as jnp
from jax import lax
from jax.experimental import pallas as pl
from jax.experimental.pallas import tpu as pltpu
from jax.experimental.pallas import tpu_sc as plsc

B = 4096
S = 50
D = 144
TD = 24
V = 100000
N_ROWS = B * S
NC, NS = 2, 16
NW = NC * NS
BW = B // NW
BB = 1024
VB = 4096


def _pre_body(tt_ref, lo_ref, hi_ref):
    x = jnp.transpose(tt_ref[...], (1, 0))
    lo_ref[...] = x[:, :128]
    hi_ref[...] = jnp.concatenate(
        [x[:, 128:], jnp.zeros((VB, 128 - (D - 128)), jnp.float32)], axis=1)


def _pre_call(tab_t):
    spec = pl.BlockSpec((VB, 128), lambda i: (i, 0))
    return pl.pallas_call(
        _pre_body,
        grid=(pl.cdiv(V, VB),),
        in_specs=[pl.BlockSpec((D, VB), lambda i: (0, i))],
        out_specs=[spec, spec],
        out_shape=[jax.ShapeDtypeStruct((V, 128), jnp.float32)] * 2,
    )(tab_t)


def _sc_body(idst_hbm, tlo_hbm, thi_hbm, lo_hbm, hi_hbm,
             idx_all, rlo0, rlo1, rhi0, rhi1, gs0, gs1, os0, os1):
    wid = lax.axis_index("s") * NC + lax.axis_index("c")
    b0 = wid * BW
    rlos = (rlo0, rlo1)
    rhis = (rhi0, rhi1)
    gsems = (gs0, gs1)
    osems = (os0, os1)

    pltpu.sync_copy(idst_hbm.at[:, pl.ds(b0, BW)], idx_all)

    def gstart(s, b):
        pltpu.async_copy(tlo_hbm.at[idx_all.at[s]], rlos[b], gsems[b])
        pltpu.async_copy(thi_hbm.at[idx_all.at[s]], rhis[b], gsems[b])

    def gwait(s, b):
        pltpu.make_async_copy(tlo_hbm.at[idx_all.at[s]], rlos[b],
                              gsems[b]).wait()
        pltpu.make_async_copy(thi_hbm.at[idx_all.at[s]], rhis[b],
                              gsems[b]).wait()

    def ostart(s, b):
        r0 = s * B + b0
        pltpu.async_copy(rlos[b], lo_hbm.at[pl.ds(r0, BW)], osems[b])
        pltpu.async_copy(rhis[b], hi_hbm.at[pl.ds(r0, BW)], osems[b])

    def owait(s, b):
        r0 = s * B + b0
        pltpu.make_async_copy(rlos[b], lo_hbm.at[pl.ds(r0, BW)],
                              osems[b]).wait()
        pltpu.make_async_copy(rhis[b], hi_hbm.at[pl.ds(r0, BW)],
                              osems[b]).wait()

    gstart(0, 0)

    def two_steps(i, carry):
        for b in (0, 1):
            s = 2 * i + b
            nb = 1 - b

            @pl.when(s >= 1)
            def _():
                owait(s - 1, nb)

            @pl.when(s + 1 < S)
            def _():
                gstart(s + 1, nb)

            gwait(s, b)
            ostart(s, b)
        return carry

    lax.fori_loop(0, S // 2, two_steps, 0)
    owait(S - 1, 1)


_mesh = plsc.VectorSubcoreMesh(core_axis_name="c", subcore_axis_name="s")
_sc_out = (
    jax.ShapeDtypeStruct((N_ROWS, 128), jnp.float32),
    jax.ShapeDtypeStruct((N_ROWS, 128), jnp.float32),
)
_sc_scratch = [
    pltpu.VMEM((S, BW), jnp.int32),
    pltpu.VMEM((BW, 128), jnp.float32),
    pltpu.VMEM((BW, 128), jnp.float32),
    pltpu.VMEM((BW, 128), jnp.float32),
    pltpu.VMEM((BW, 128), jnp.float32),
    pltpu.SemaphoreType.DMA,
    pltpu.SemaphoreType.DMA,
    pltpu.SemaphoreType.DMA,
    pltpu.SemaphoreType.DMA,
]
_sc_call = pl.kernel(_sc_body, out_type=_sc_out, mesh=_mesh,
                     scratch_types=_sc_scratch,
                     compiler_params=pltpu.CompilerParams(
                         use_tc_tiling_on_sc=False))


def _tc_body(lo_ref, hi_ref, x_ref, *out_refs):
    lo = lo_ref[...]
    hi = hi_ref[...]
    loT = jnp.transpose(lo, (1, 0))
    hiT = jnp.transpose(hi, (1, 0))
    raw = jnp.concatenate([loT, hiT[:16]], axis=0)
    x_ref[0, :, :] = raw
    sp = jnp.maximum(raw[:48], 0.0) + jnp.log1p(jnp.exp(-jnp.abs(raw[:48])))
    sg = jax.nn.sigmoid(raw[48:])
    act = jnp.concatenate([sp, sg], axis=0)
    for k, o in enumerate(out_refs):
        o[0, :, :] = act[TD * k:TD * (k + 1), :]


def _tc_call(lo, hi):
    grid = (S, B // BB)
    in_spec = pl.BlockSpec((BB, 128), lambda s, j: (s * (B // BB) + j, 0))
    x_spec = pl.BlockSpec((1, D, BB), lambda s, j: (s, 0, j))
    a_spec = pl.BlockSpec((1, TD, BB), lambda s, j: (s, 0, j))
    return pl.pallas_call(
        _tc_body,
        grid=grid,
        in_specs=[in_spec, in_spec],
        out_specs=[x_spec] + [a_spec] * 6,
        out_shape=[jax.ShapeDtypeStruct((S, D, B), jnp.float32)] +
                  [jax.ShapeDtypeStruct((S, TD, B), jnp.float32)] * 6,
    )(lo, hi)


def kernel(quant_ids, quant_embedding_weight):
    ids_t = jnp.transpose(quant_ids).astype(jnp.int32)
    tab_lo, tab_hi = _pre_call(jnp.transpose(quant_embedding_weight))
    lo, hi = _sc_call(ids_t, tab_lo, tab_hi)
    x_t, qs, dz, ts, rs, th, rh = _tc_call(lo, hi)
    tr = lambda a: jnp.transpose(a, (2, 0, 1))
    return (tr(x_t), tr(qs), tr(dz), tr(rh), tr(th), tr(rs), tr(ts))


# speedup vs baseline: 14.4347x; 1.1805x over previous
import jax
import jax.numpy as jnp
from jax import lax
from jax.experimental import pallas as pl
from jax.experimental.pallas import tpu as pltpu
from jax.experimental.pallas import tpu_sc as plsc

B = 4096
S = 50
D = 144
TD = 24
V = 100000
N_ROWS = B * S
NC, NS = 2, 16
NW = NC * NS
BW = B // NW
BB = 2048
VB = 8192


def _pre_body(tt_ref, lo_ref, hi_ref):
    x = jnp.transpose(tt_ref[...], (1, 0))
    lo_ref[...] = x[:, :128]
    hi_ref[...] = jnp.concatenate(
        [x[:, 128:], jnp.zeros((VB, 128 - (D - 128)), jnp.float32)], axis=1)


def _pre_call(tab_t):
    spec = pl.BlockSpec((VB, 128), lambda i: (i, 0))
    return pl.pallas_call(
        _pre_body,
        grid=(pl.cdiv(V, VB),),
        in_specs=[pl.BlockSpec((D, VB), lambda i: (0, i))],
        out_specs=[spec, spec],
        out_shape=[jax.ShapeDtypeStruct((V, 128), jnp.float32)] * 2,
    )(tab_t)


def _sc_body(idst_hbm, tlo_hbm, thi_hbm, lo_hbm, hi_hbm,
             idx_all, rlo0, rlo1, rhi0, rhi1, gs0, gs1, os0, os1):
    wid = lax.axis_index("s") * NC + lax.axis_index("c")
    b0 = wid * BW
    rlos = (rlo0, rlo1)
    rhis = (rhi0, rhi1)
    gsems = (gs0, gs1)
    osems = (os0, os1)

    pltpu.sync_copy(idst_hbm.at[:, pl.ds(b0, BW)], idx_all)

    def gstart(s, b):
        pltpu.async_copy(tlo_hbm.at[idx_all.at[s]], rlos[b], gsems[b])
        pltpu.async_copy(thi_hbm.at[idx_all.at[s]], rhis[b], gsems[b])

    def gwait(s, b):
        pltpu.make_async_copy(tlo_hbm.at[idx_all.at[s]], rlos[b],
                              gsems[b]).wait()
        pltpu.make_async_copy(thi_hbm.at[idx_all.at[s]], rhis[b],
                              gsems[b]).wait()

    def ostart(s, b):
        r0 = s * B + b0
        pltpu.async_copy(rlos[b], lo_hbm.at[pl.ds(r0, BW)], osems[b])
        pltpu.async_copy(rhis[b], hi_hbm.at[pl.ds(r0, BW)], osems[b])

    def owait(s, b):
        r0 = s * B + b0
        pltpu.make_async_copy(rlos[b], lo_hbm.at[pl.ds(r0, BW)],
                              osems[b]).wait()
        pltpu.make_async_copy(rhis[b], hi_hbm.at[pl.ds(r0, BW)],
                              osems[b]).wait()

    gstart(0, 0)

    def two_steps(i, carry):
        for b in (0, 1):
            s = 2 * i + b
            nb = 1 - b

            @pl.when(s >= 1)
            def _():
                owait(s - 1, nb)

            @pl.when(s + 1 < S)
            def _():
                gstart(s + 1, nb)

            gwait(s, b)
            ostart(s, b)
        return carry

    lax.fori_loop(0, S // 2, two_steps, 0)
    owait(S - 1, 1)


_mesh = plsc.VectorSubcoreMesh(core_axis_name="c", subcore_axis_name="s")
_sc_out = (
    jax.ShapeDtypeStruct((N_ROWS, 128), jnp.float32),
    jax.ShapeDtypeStruct((N_ROWS, 128), jnp.float32),
)
_sc_scratch = [
    pltpu.VMEM((S, BW), jnp.int32),
    pltpu.VMEM((BW, 128), jnp.float32),
    pltpu.VMEM((BW, 128), jnp.float32),
    pltpu.VMEM((BW, 128), jnp.float32),
    pltpu.VMEM((BW, 128), jnp.float32),
    pltpu.SemaphoreType.DMA,
    pltpu.SemaphoreType.DMA,
    pltpu.SemaphoreType.DMA,
    pltpu.SemaphoreType.DMA,
]
_sc_call = pl.kernel(_sc_body, out_type=_sc_out, mesh=_mesh,
                     scratch_types=_sc_scratch,
                     compiler_params=pltpu.CompilerParams(
                         use_tc_tiling_on_sc=False))


def _tc_body(lo_ref, hi_ref, x_ref, *out_refs):
    lo = lo_ref[...]
    hi = hi_ref[...]
    loT = jnp.transpose(lo, (1, 0))
    hiT = jnp.transpose(hi, (1, 0))
    raw = jnp.concatenate([loT, hiT[:16]], axis=0)
    x_ref[0, :, :] = raw
    sp = jnp.maximum(raw[:48], 0.0) + jnp.log1p(jnp.exp(-jnp.abs(raw[:48])))
    sg = jax.nn.sigmoid(raw[48:])
    act = jnp.concatenate([sp, sg], axis=0)
    for k, o in enumerate(out_refs):
        o[0, :, :] = act[TD * k:TD * (k + 1), :]


def _tc_call(lo, hi):
    grid = (S, B // BB)
    in_spec = pl.BlockSpec((BB, 128), lambda s, j: (s * (B // BB) + j, 0))
    x_spec = pl.BlockSpec((1, D, BB), lambda s, j: (s, 0, j))
    a_spec = pl.BlockSpec((1, TD, BB), lambda s, j: (s, 0, j))
    return pl.pallas_call(
        _tc_body,
        grid=grid,
        in_specs=[in_spec, in_spec],
        out_specs=[x_spec] + [a_spec] * 6,
        out_shape=[jax.ShapeDtypeStruct((S, D, B), jnp.float32)] +
                  [jax.ShapeDtypeStruct((S, TD, B), jnp.float32)] * 6,
    )(lo, hi)


def kernel(quant_ids, quant_embedding_weight):
    ids_t = jnp.transpose(quant_ids).astype(jnp.int32)
    tab_lo, tab_hi = _pre_call(jnp.transpose(quant_embedding_weight))
    lo, hi = _sc_call(ids_t, tab_lo, tab_hi)
    x_t, qs, dz, ts, rs, th, rh = _tc_call(lo, hi)
    tr = lambda a: jnp.transpose(a, (2, 0, 1))
    return (tr(x_t), tr(qs), tr(dz), tr(rh), tr(th), tr(rs), tr(ts))


# speedup vs baseline: 15.2544x; 1.0568x over previous
import jax
import jax.numpy as jnp
from jax import lax
from jax.experimental import pallas as pl
from jax.experimental.pallas import tpu as pltpu
from jax.experimental.pallas import tpu_sc as plsc

B = 4096
S = 50
D = 144
TD = 24
V = 100000
N_ROWS = B * S
NC, NS = 2, 16
NW = NC * NS
BW = B // NW
BB = 4096
VB = 8192


def _pre_body(tt_ref, lo_ref, hi_ref):
    x = jnp.transpose(tt_ref[...], (1, 0))
    lo_ref[...] = x[:, :128]
    hi_ref[...] = jnp.concatenate(
        [x[:, 128:], jnp.zeros((VB, 128 - (D - 128)), jnp.float32)], axis=1)


def _pre_call(tab_t):
    spec = pl.BlockSpec((VB, 128), lambda i: (i, 0))
    return pl.pallas_call(
        _pre_body,
        grid=(pl.cdiv(V, VB),),
        in_specs=[pl.BlockSpec((D, VB), lambda i: (0, i))],
        out_specs=[spec, spec],
        out_shape=[jax.ShapeDtypeStruct((V, 128), jnp.float32)] * 2,
    )(tab_t)


def _sc_body(idst_hbm, tlo_hbm, thi_hbm, lo_hbm, hi_hbm,
             idx_all, rlo0, rlo1, rhi0, rhi1, gs0, gs1, os0, os1):
    wid = lax.axis_index("s") * NC + lax.axis_index("c")
    b0 = wid * BW
    rlos = (rlo0, rlo1)
    rhis = (rhi0, rhi1)
    gsems = (gs0, gs1)
    osems = (os0, os1)

    pltpu.sync_copy(idst_hbm.at[:, pl.ds(b0, BW)], idx_all)

    def gstart(s, b):
        pltpu.async_copy(tlo_hbm.at[idx_all.at[s]], rlos[b], gsems[b])
        pltpu.async_copy(thi_hbm.at[idx_all.at[s]], rhis[b], gsems[b])

    def gwait(s, b):
        pltpu.make_async_copy(tlo_hbm.at[idx_all.at[s]], rlos[b],
                              gsems[b]).wait()
        pltpu.make_async_copy(thi_hbm.at[idx_all.at[s]], rhis[b],
                              gsems[b]).wait()

    def ostart(s, b):
        r0 = s * B + b0
        pltpu.async_copy(rlos[b], lo_hbm.at[pl.ds(r0, BW)], osems[b])
        pltpu.async_copy(rhis[b], hi_hbm.at[pl.ds(r0, BW)], osems[b])

    def owait(s, b):
        r0 = s * B + b0
        pltpu.make_async_copy(rlos[b], lo_hbm.at[pl.ds(r0, BW)],
                              osems[b]).wait()
        pltpu.make_async_copy(rhis[b], hi_hbm.at[pl.ds(r0, BW)],
                              osems[b]).wait()

    gstart(0, 0)

    def two_steps(i, carry):
        for b in (0, 1):
            s = 2 * i + b
            nb = 1 - b

            @pl.when(s >= 1)
            def _():
                owait(s - 1, nb)

            @pl.when(s + 1 < S)
            def _():
                gstart(s + 1, nb)

            gwait(s, b)
            ostart(s, b)
        return carry

    lax.fori_loop(0, S // 2, two_steps, 0)
    owait(S - 1, 1)


_mesh = plsc.VectorSubcoreMesh(core_axis_name="c", subcore_axis_name="s")
_sc_out = (
    jax.ShapeDtypeStruct((N_ROWS, 128), jnp.float32),
    jax.ShapeDtypeStruct((N_ROWS, 128), jnp.float32),
)
_sc_scratch = [
    pltpu.VMEM((S, BW), jnp.int32),
    pltpu.VMEM((BW, 128), jnp.float32),
    pltpu.VMEM((BW, 128), jnp.float32),
    pltpu.VMEM((BW, 128), jnp.float32),
    pltpu.VMEM((BW, 128), jnp.float32),
    pltpu.SemaphoreType.DMA,
    pltpu.SemaphoreType.DMA,
    pltpu.SemaphoreType.DMA,
    pltpu.SemaphoreType.DMA,
]
_sc_call = pl.kernel(_sc_body, out_type=_sc_out, mesh=_mesh,
                     scratch_types=_sc_scratch,
                     compiler_params=pltpu.CompilerParams(
                         use_tc_tiling_on_sc=False))


def _tc_body(lo_ref, hi_ref, x_ref, *out_refs):
    lo = lo_ref[...]
    hi = hi_ref[...]
    loT = jnp.transpose(lo, (1, 0))
    hiT = jnp.transpose(hi, (1, 0))
    raw = jnp.concatenate([loT, hiT[:16]], axis=0)
    x_ref[0, :, :] = raw
    sp = jnp.maximum(raw[:48], 0.0) + jnp.log1p(jnp.exp(-jnp.abs(raw[:48])))
    sg = jax.nn.sigmoid(raw[48:])
    act = jnp.concatenate([sp, sg], axis=0)
    for k, o in enumerate(out_refs):
        o[0, :, :] = act[TD * k:TD * (k + 1), :]


def _tc_call(lo, hi):
    grid = (S, B // BB)
    in_spec = pl.BlockSpec((BB, 128), lambda s, j: (s * (B // BB) + j, 0))
    x_spec = pl.BlockSpec((1, D, BB), lambda s, j: (s, 0, j))
    a_spec = pl.BlockSpec((1, TD, BB), lambda s, j: (s, 0, j))
    return pl.pallas_call(
        _tc_body,
        grid=grid,
        in_specs=[in_spec, in_spec],
        out_specs=[x_spec] + [a_spec] * 6,
        out_shape=[jax.ShapeDtypeStruct((S, D, B), jnp.float32)] +
                  [jax.ShapeDtypeStruct((S, TD, B), jnp.float32)] * 6,
    )(lo, hi)


def kernel(quant_ids, quant_embedding_weight):
    ids_t = jnp.transpose(quant_ids).astype(jnp.int32)
    tab_lo, tab_hi = _pre_call(jnp.transpose(quant_embedding_weight))
    lo, hi = _sc_call(ids_t, tab_lo, tab_hi)
    x_t, qs, dz, ts, rs, th, rh = _tc_call(lo, hi)
    tr = lambda a: jnp.transpose(a, (2, 0, 1))
    return (tr(x_t), tr(qs), tr(dz), tr(rh), tr(th), tr(rs), tr(ts))
